# Initial kernel scaffold; baseline (speedup 1.0000x reference)
import functools

import jax
import jax.numpy as jnp
from jax import lax
from jax.experimental import pallas as pl
from jax.experimental.pallas import tpu as pltpu
from jax.experimental.pallas import tpu_sc as plsc

EMBED_DIM = 32
NUM_CORES = 2
NUM_SUBCORES = 16
NUM_WORKERS = NUM_CORES * NUM_SUBCORES
CHUNK = 1600


def _gather_kernel(idx_hbm, table_hbm, out_hbm, idx_v, rows_v, sem, *,
                   b_per_w, n_chunks):
    wid = lax.axis_index("s") * NUM_CORES + lax.axis_index("c")
    base = wid * b_per_w

    def body(i, carry):
        off = base + i * CHUNK
        pltpu.sync_copy(idx_hbm.at[pl.ds(off, CHUNK)], idx_v)
        pltpu.async_copy(table_hbm.at[idx_v], rows_v, sem).wait()
        pltpu.sync_copy(rows_v, out_hbm.at[pl.ds(off, CHUNK)])
        return carry

    lax.fori_loop(0, n_chunks, body, 0)


def kernel(input, weight):
    b0, b1 = input.shape
    total = b0 * b1
    assert total % (NUM_WORKERS * CHUNK) == 0
    b_per_w = total // NUM_WORKERS
    n_chunks = b_per_w // CHUNK

    idx_flat = input.reshape(total).astype(jnp.int32)

    mesh = plsc.VectorSubcoreMesh(core_axis_name="c", subcore_axis_name="s")
    run = pl.kernel(
        functools.partial(_gather_kernel, b_per_w=b_per_w, n_chunks=n_chunks),
        mesh=mesh,
        out_type=jax.ShapeDtypeStruct((total, EMBED_DIM), jnp.float32),
        scratch_types=[
            pltpu.VMEM((CHUNK,), jnp.int32),
            pltpu.VMEM((CHUNK, EMBED_DIM), jnp.float32),
            pltpu.SemaphoreType.DMA,
        ],
    )
    out = run(idx_flat, weight)
    return out.reshape(b0, b1, EMBED_DIM)


# speedup vs baseline: 1.1029x; 1.1029x over previous
import functools

import jax
import jax.numpy as jnp
from jax import lax
from jax.experimental import pallas as pl
from jax.experimental.pallas import tpu as pltpu
from jax.experimental.pallas import tpu_sc as plsc

EMBED_DIM = 32
NUM_CORES = 2
NUM_SUBCORES = 16
NUM_WORKERS = NUM_CORES * NUM_SUBCORES
CHUNK = 1600


def _gather_kernel(idx_hbm, table_hbm, out_hbm, idx_v, rows_v, sem, *,
                   b_per_w, n_chunks):
    wid = lax.axis_index("s") * NUM_CORES + lax.axis_index("c")
    base = wid * b_per_w

    def body(i, carry):
        off = base + i * CHUNK
        pltpu.sync_copy(idx_hbm.at[pl.ds(off, CHUNK)], idx_v)
        pltpu.async_copy(table_hbm.at[idx_v], rows_v, sem).wait()
        pltpu.sync_copy(rows_v, out_hbm.at[pl.ds(off, CHUNK)])
        return carry

    lax.fori_loop(0, n_chunks, body, 0)


def kernel(input, weight):
    b0, b1 = input.shape
    total = b0 * b1
    assert total % (NUM_WORKERS * CHUNK) == 0
    b_per_w = total // NUM_WORKERS
    n_chunks = b_per_w // CHUNK

    idx_flat = input.reshape(total).astype(jnp.int32)

    mesh = plsc.VectorSubcoreMesh(core_axis_name="c", subcore_axis_name="s")
    run = pl.kernel(
        functools.partial(_gather_kernel, b_per_w=b_per_w, n_chunks=n_chunks),
        mesh=mesh,
        out_type=jax.ShapeDtypeStruct((total, EMBED_DIM), jnp.float32),
        scratch_types=[
            pltpu.VMEM((CHUNK,), jnp.int32),
            pltpu.VMEM((CHUNK, EMBED_DIM), jnp.float32),
            pltpu.SemaphoreType.DMA,
        ],
        compiler_params=pltpu.CompilerParams(use_tc_tiling_on_sc=False),
    )
    out = run(idx_flat, weight)
    return out.reshape(b0, b1, EMBED_DIM)


# speedup vs baseline: 1.1097x; 1.0061x over previous
import functools

import jax
import jax.numpy as jnp
from jax import lax
from jax.experimental import pallas as pl
from jax.experimental.pallas import tpu as pltpu
from jax.experimental.pallas import tpu_sc as plsc

EMBED_DIM = 32
NUM_CORES = 2
NUM_SUBCORES = 16
NUM_WORKERS = NUM_CORES * NUM_SUBCORES
CHUNK = 1600
NBUF = 2


def _gather_kernel(idx_hbm, table_hbm, out_hbm, idx_v, rows_v, isem, gsem,
                   wsem, *, b_per_w, n_chunks):
    wid = lax.axis_index("s") * NUM_CORES + lax.axis_index("c")
    base = wid * b_per_w

    pltpu.async_copy(idx_hbm.at[pl.ds(base, CHUNK)], idx_v.at[0], isem.at[0])

    def body(g, carry):
        for b in range(NBUF):
            i = g * NBUF + b

            @pl.when(i >= NBUF)
            def _():
                pltpu.make_async_copy(
                    rows_v.at[b], out_hbm.at[pl.ds(base, CHUNK)], wsem.at[b]
                ).wait()

            pltpu.make_async_copy(
                idx_hbm.at[pl.ds(base, CHUNK)], idx_v.at[b], isem.at[b]
            ).wait()

            gather = pltpu.async_copy(
                table_hbm.at[idx_v.at[b]], rows_v.at[b], gsem.at[b]
            )

            @pl.when(i + 1 < n_chunks)
            def _():
                pltpu.async_copy(
                    idx_hbm.at[pl.ds(base + (i + 1) * CHUNK, CHUNK)],
                    idx_v.at[1 - b],
                    isem.at[1 - b],
                )

            gather.wait()

            pltpu.async_copy(
                rows_v.at[b],
                out_hbm.at[pl.ds(base + i * CHUNK, CHUNK)],
                wsem.at[b],
            )
        return carry

    lax.fori_loop(0, n_chunks // NBUF, body, 0)

    for b in range(NBUF):
        pltpu.make_async_copy(
            rows_v.at[b], out_hbm.at[pl.ds(base, CHUNK)], wsem.at[b]
        ).wait()


def kernel(input, weight):
    b0, b1 = input.shape
    total = b0 * b1
    assert total % (NUM_WORKERS * CHUNK * NBUF) == 0
    b_per_w = total // NUM_WORKERS
    n_chunks = b_per_w // CHUNK

    idx_flat = input.reshape(total).astype(jnp.int32)

    mesh = plsc.VectorSubcoreMesh(core_axis_name="c", subcore_axis_name="s")
    run = pl.kernel(
        functools.partial(_gather_kernel, b_per_w=b_per_w, n_chunks=n_chunks),
        mesh=mesh,
        out_type=jax.ShapeDtypeStruct((total, EMBED_DIM), jnp.float32),
        scratch_types=[
            pltpu.VMEM((NBUF, CHUNK), jnp.int32),
            pltpu.VMEM((NBUF, CHUNK, EMBED_DIM), jnp.float32),
            pltpu.SemaphoreType.DMA((NBUF,)),
            pltpu.SemaphoreType.DMA((NBUF,)),
            pltpu.SemaphoreType.DMA((NBUF,)),
        ],
        compiler_params=pltpu.CompilerParams(use_tc_tiling_on_sc=False),
    )
    out = run(idx_flat, weight)
    return out.reshape(b0, b1, EMBED_DIM)


# speedup vs baseline: 1.6261x; 1.4653x over previous
import functools

import jax
import jax.numpy as jnp
from jax import lax
from jax.experimental import pallas as pl
from jax.experimental.pallas import tpu as pltpu
from jax.experimental.pallas import tpu_sc as plsc

EMBED_DIM = 32
NUM_CORES = 2
NUM_SUBCORES = 16
NUM_WORKERS = NUM_CORES * NUM_SUBCORES
B0 = 16384
B1 = 50
COLS = B0 // NUM_WORKERS
NBLK = COLS // 128


def _embed_kernel(idx_hbm, table_hbm, out_hbm, idx_blk, idx_list, rows_v,
                  t_v, isem, gsem, wsem):
    wid = lax.axis_index("s") * NUM_CORES + lax.axis_index("c")
    col0 = wid * COLS
    viota = lax.iota(jnp.int32, 16)

    def extract_idx(b1, buf):
        for j in range(COLS // 16):
            rows = viota + (16 * j)
            cols = jnp.full((16,), b1, jnp.int32)
            vals = plsc.load_gather(idx_blk, [rows, cols])
            idx_list[buf, pl.ds(16 * j, 16)] = vals

    def issue_gather(buf):
        return pltpu.async_copy(
            table_hbm.at[idx_list.at[buf]], rows_v.at[buf], gsem.at[buf]
        )

    def wait_gather(buf):
        pltpu.make_async_copy(
            table_hbm.at[idx_list.at[buf]], rows_v.at[buf], gsem.at[buf]
        ).wait()

    def transpose(buf):
        def tbody(fb_bb, carry):
            f_blk = fb_bb // NBLK
            bb = fb_bb % NBLK
            rows2d = rows_v.at[buf]
            for f8 in range(8):
                cols = jnp.full((16,), 0, jnp.int32) + (f_blk * 8 + f8)
                for jl in range(8):
                    rows = viota + (bb * 128 + jl * 16)
                    vals = plsc.load_gather(rows2d, [rows, cols])
                    t_v[buf, f_blk, bb, f8, pl.ds(jl * 16, 16)] = vals
            return carry

        lax.fori_loop(0, 4 * NBLK, tbody, 0)

    def wait_wb(buf):
        pltpu.make_async_copy(
            t_v.at[buf], out_hbm.at[0, :, pl.ds(0, NBLK), :, :], wsem.at[buf]
        ).wait()

    pltpu.async_copy(
        idx_hbm.at[pl.ds(col0, COLS), :], idx_blk, isem
    ).wait()

    extract_idx(0, 0)
    issue_gather(0)

    def body(g, carry):
        for buf in range(2):
            i = g * 2 + buf
            wait_gather(buf)

            @pl.when(i + 1 < B1)
            def _():
                extract_idx(i + 1, 1 - buf)
                issue_gather(1 - buf)

            @pl.when(i >= 2)
            def _():
                wait_wb(buf)

            transpose(buf)
            pltpu.async_copy(
                t_v.at[buf],
                out_hbm.at[i, :, pl.ds(wid * NBLK, NBLK), :, :],
                wsem.at[buf],
            )
        return carry

    lax.fori_loop(0, B1 // 2, body, 0)
    for buf in range(2):
        wait_wb(buf)


def kernel(input, weight):
    idx = input.astype(jnp.int32)

    mesh = plsc.VectorSubcoreMesh(core_axis_name="c", subcore_axis_name="s")
    run = pl.kernel(
        _embed_kernel,
        mesh=mesh,
        out_type=jax.ShapeDtypeStruct((B1, 4, B0 // 128, 8, 128), jnp.float32),
        scratch_types=[
            pltpu.VMEM((COLS, B1), jnp.int32),
            pltpu.VMEM((2, COLS), jnp.int32),
            pltpu.VMEM((2, COLS, EMBED_DIM), jnp.float32),
            pltpu.VMEM((2, 4, NBLK, 8, 128), jnp.float32),
            pltpu.SemaphoreType.DMA,
            pltpu.SemaphoreType.DMA((2,)),
            pltpu.SemaphoreType.DMA((2,)),
        ],
        compiler_params=pltpu.CompilerParams(
            use_tc_tiling_on_sc=False, needs_layout_passes=False
        ),
    )
    out5d = run(idx, weight)
    return out5d.transpose(2, 4, 0, 1, 3).reshape(B0, B1, EMBED_DIM)
